# Initial kernel scaffold; baseline (speedup 1.0000x reference)
import jax
import jax.numpy as jnp
from jax.experimental import pallas as pl


def kernel(x, table):
    raise NotImplementedError("write your pallas kernel here")


# speedup vs baseline: 2.2960x; 2.2960x over previous
import jax
import jax.numpy as jnp
from jax.experimental import pallas as pl


def _bcast_body(t_ref, o_ref):
    o_ref[...] = jnp.broadcast_to(t_ref[...][None], o_ref.shape)


def kernel(x, table):
    batch = x.shape[0]
    seq, dim = table.shape
    bs = 512
    out = pl.pallas_call(
        _bcast_body,
        grid=(seq // bs,),
        in_specs=[pl.BlockSpec((bs, dim), lambda s: (s, 0))],
        out_specs=pl.BlockSpec((batch, bs, dim), lambda s: (0, s, 0)),
        out_shape=jax.ShapeDtypeStruct((batch, seq, dim), table.dtype),
    )(table)
    return out
